# baseline (speedup 1.0000x reference)
import functools

import jax
import jax.numpy as jnp
from jax import lax
from jax.experimental import pallas as pl
from jax.experimental.pallas import tpu as pltpu
from jax.experimental.pallas import tpu_sc as plsc

VOCAB = 1000000
BATCH = 4096
SEQ = 200
SEQ_PAD = 256
EMBED_DIM = 32
NUM_CLASSES = 100
CLASS_PAD = 128

NUM_CORES = 2
NUM_SUBCORES = 16
NUM_WORKERS = NUM_CORES * NUM_SUBCORES
BPW = BATCH // NUM_WORKERS

_TPS = 128
_NBUF = 4
_GMAX = BPW // 2


def _sc_pool_sums(x_pad, table4):
    mesh = plsc.VectorSubcoreMesh(core_axis_name="c", subcore_axis_name="s")

    @functools.partial(
        pl.kernel,
        mesh=mesh,
        compiler_params=pltpu.CompilerParams(use_tc_tiling_on_sc=False),
        out_type=jax.ShapeDtypeStruct((BATCH, EMBED_DIM), jnp.float32),
        scratch_types=[
            pltpu.VMEM((BPW, SEQ_PAD), jnp.int32),
            pltpu.VMEM((_NBUF, _TPS), jnp.int32),
            pltpu.VMEM((_NBUF, _TPS, 128), jnp.float32),
            pltpu.VMEM((BPW, EMBED_DIM), jnp.float32),
            pltpu.SemaphoreType.DMA,
            pltpu.SemaphoreType.DMA,
            pltpu.SemaphoreType.DMA,
            pltpu.SemaphoreType.DMA,
        ],
    )
    def body(x_hbm, table_hbm, out_hbm, idx_v, gidx_v, rows_v, sums_v, *sems):
        wid = lax.axis_index("s") * NUM_CORES + lax.axis_index("c")
        base = wid * BPW
        pltpu.sync_copy(x_hbm.at[pl.ds(base, BPW)], idx_v)

        def issue(g, j):
            row = g * 2 + (j >> 1)
            t0 = (j & 1) * _TPS
            for c in range(_TPS // 16):
                v = idx_v[row, pl.ds(t0 + c * 16, 16)]
                gidx_v[j, pl.ds(c * 16, 16)] = v >> 2
            pltpu.async_copy(
                table_hbm.at[gidx_v.at[j]], rows_v.at[j], sems[j])

        def drain(j):
            pltpu.make_async_copy(
                table_hbm.at[pl.ds(0, _TPS)], rows_v.at[j], sems[j]).wait()

        for j in range(_NBUF):
            issue(0, j)

        zero = jnp.zeros((16,), jnp.float32)

        def outer(g, carry):
            for jr in range(2):
                row = g * 2 + jr
                accs = (zero,) * 8
                for jh in range(2):
                    j = jr * 2 + jh
                    t0 = jh * _TPS
                    drain(j)

                    def tok_chunk(c, accs, _j=j, _t0=t0, _row=row):
                        accs = list(accs)
                        k0 = c * 16
                        ivv = idx_v[_row, pl.ds(_t0 + k0, 16)]
                        colv = (ivv & 3) * 32
                        for u in range(16):
                            col = colv[u]
                            p = (u % 4) * 2
                            accs[p] = accs[p] + rows_v[
                                _j, k0 + u, pl.ds(col, 16)]
                            accs[p + 1] = accs[p + 1] + rows_v[
                                _j, k0 + u, pl.ds(col + 16, 16)]
                        return tuple(accs)

                    accs = lax.fori_loop(0, _TPS // 16, tok_chunk, accs)

                    @pl.when(g < _GMAX - 1)
                    def _(_g=g, _j=j):
                        issue(_g + 1, _j)

                s0 = (accs[0] + accs[2]) + (accs[4] + accs[6])
                s1 = (accs[1] + accs[3]) + (accs[5] + accs[7])
                sums_v[row, pl.ds(0, 16)] = s0
                sums_v[row, pl.ds(16, 16)] = s1
            return carry

        lax.fori_loop(0, _GMAX, outer, 0)
        pltpu.sync_copy(sums_v, out_hbm.at[pl.ds(base, BPW)])

    return body(x_pad, table4)


def _tc_matmul(sums, x_pad, t0, w_scaled, b_pad):
    def body(s_ref, x_ref, t0_ref, w_ref, b_ref, o_ref):
        cnt0 = jnp.sum((x_ref[...] == 0).astype(jnp.float32), axis=1,
                       keepdims=True)
        pooled = s_ref[...] - cnt0 * t0_ref[...]
        o_ref[...] = jnp.dot(
            pooled, w_ref[...], preferred_element_type=jnp.float32
        ) + b_ref[...]

    blk = 1024
    return pl.pallas_call(
        body,
        grid=(BATCH // blk,),
        in_specs=[
            pl.BlockSpec((blk, EMBED_DIM), lambda i: (i, 0)),
            pl.BlockSpec((blk, SEQ_PAD), lambda i: (i, 0)),
            pl.BlockSpec((1, EMBED_DIM), lambda i: (0, 0)),
            pl.BlockSpec((EMBED_DIM, CLASS_PAD), lambda i: (0, 0)),
            pl.BlockSpec((1, CLASS_PAD), lambda i: (0, 0)),
        ],
        out_specs=pl.BlockSpec((blk, CLASS_PAD), lambda i: (i, 0)),
        out_shape=jax.ShapeDtypeStruct((BATCH, CLASS_PAD), jnp.float32),
    )(sums, x_pad, t0, w_scaled, b_pad)


def kernel(x, table, W, b):
    x_pad = jnp.pad(x, ((0, 0), (0, SEQ_PAD - SEQ)))
    table4 = jnp.reshape(table, (VOCAB // 4, 128))
    t0 = lax.slice(table, (0, 0), (1, EMBED_DIM))
    w_scaled = jnp.zeros((EMBED_DIM, CLASS_PAD), jnp.float32)
    w_scaled = w_scaled.at[:, :NUM_CLASSES].set(W.T * (1.0 / SEQ))
    b_pad = jnp.zeros((1, CLASS_PAD), jnp.float32).at[0, :NUM_CLASSES].set(b)

    sums = _sc_pool_sums(x_pad, table4)
    logits_pad = _tc_matmul(sums, x_pad, t0, w_scaled, b_pad)
    return logits_pad[:, :NUM_CLASSES]


# speedup vs baseline: 10.6504x; 10.6504x over previous
import functools

import jax
import jax.numpy as jnp
from jax import lax
from jax.experimental import pallas as pl
from jax.experimental.pallas import tpu as pltpu
from jax.experimental.pallas import tpu_sc as plsc

BATCH = 4096
SEQ = 200
SEQ_PAD = 208
EMBED_DIM = 32
NUM_CLASSES = 100
CLASS_PAD = 128

NUM_CORES = 2
NUM_SUBCORES = 16
NUM_WORKERS = NUM_CORES * NUM_SUBCORES
BPW = BATCH // NUM_WORKERS

_N16 = SEQ_PAD // 16
_G1 = 128
_G2 = SEQ_PAD - _G1
_NBUF = 8


def _sc_pool_sums(x_pad, table):
    mesh = plsc.VectorSubcoreMesh(core_axis_name="c", subcore_axis_name="s")

    @functools.partial(
        pl.kernel,
        mesh=mesh,
        compiler_params=pltpu.CompilerParams(use_tc_tiling_on_sc=False),
        out_type=jax.ShapeDtypeStruct((BATCH, EMBED_DIM), jnp.float32),
        scratch_types=[
            pltpu.VMEM((BPW, SEQ_PAD), jnp.int32),
            pltpu.VMEM((_NBUF, SEQ_PAD, EMBED_DIM), jnp.float32),
            pltpu.VMEM((BPW, EMBED_DIM), jnp.float32),
        ] + [pltpu.SemaphoreType.DMA] * _NBUF,
    )
    def body(x_hbm, table_hbm, out_hbm, idx_v, rows_v, sums_v, *sems):
        wid = lax.axis_index("s") * NUM_CORES + lax.axis_index("c")
        base = wid * BPW
        pltpu.sync_copy(x_hbm.at[pl.ds(base, BPW)], idx_v)

        def issue(row, j):
            pltpu.async_copy(
                table_hbm.at[idx_v.at[row, pl.ds(0, _G1)]],
                rows_v.at[j, pl.ds(0, _G1)], sems[j])
            pltpu.async_copy(
                table_hbm.at[idx_v.at[row, pl.ds(_G1, _G2)]],
                rows_v.at[j, pl.ds(_G1, _G2)], sems[j])

        def drain(j):
            pltpu.make_async_copy(
                table_hbm.at[pl.ds(0, SEQ_PAD)], rows_v.at[j], sems[j]).wait()

        for j in range(_NBUF):
            issue(j, j)

        zero = jnp.zeros((16,), jnp.float32)

        def outer(g, carry):
            for j in range(_NBUF):
                row = g * _NBUF + j
                drain(j)

                def chunk(c, accs):
                    a0, a1, a2, a3, a4, a5, a6, a7 = accs
                    r0 = c * 16
                    for u in range(0, 16, 4):
                        a0 = a0 + rows_v[j, r0 + u, pl.ds(0, 16)]
                        a1 = a1 + rows_v[j, r0 + u, pl.ds(16, 16)]
                        a2 = a2 + rows_v[j, r0 + u + 1, pl.ds(0, 16)]
                        a3 = a3 + rows_v[j, r0 + u + 1, pl.ds(16, 16)]
                        a4 = a4 + rows_v[j, r0 + u + 2, pl.ds(0, 16)]
                        a5 = a5 + rows_v[j, r0 + u + 2, pl.ds(16, 16)]
                        a6 = a6 + rows_v[j, r0 + u + 3, pl.ds(0, 16)]
                        a7 = a7 + rows_v[j, r0 + u + 3, pl.ds(16, 16)]
                    return (a0, a1, a2, a3, a4, a5, a6, a7)

                accs = lax.fori_loop(0, _N16, chunk, (zero,) * 8)

                nxt = row + _NBUF

                @pl.when(nxt < BPW)
                def _():
                    issue(nxt, j)

                s0 = (accs[0] + accs[2]) + (accs[4] + accs[6])
                s1 = (accs[1] + accs[3]) + (accs[5] + accs[7])
                sums_v[row, pl.ds(0, 16)] = s0
                sums_v[row, pl.ds(16, 16)] = s1
            return carry

        lax.fori_loop(0, BPW // _NBUF, outer, 0)
        pltpu.sync_copy(sums_v, out_hbm.at[pl.ds(base, BPW)])

    return body(x_pad, table)


def _tc_matmul(sums, x_pad, t0, w_scaled, b_pad):
    def body(s_ref, x_ref, t0_ref, w_ref, b_ref, o_ref):
        cnt0 = jnp.sum((x_ref[...] == 0).astype(jnp.float32), axis=1,
                       keepdims=True)
        pooled = s_ref[...] - cnt0 * t0_ref[...]
        o_ref[...] = jnp.dot(
            pooled, w_ref[...], preferred_element_type=jnp.float32
        ) + b_ref[...]

    blk = 1024
    return pl.pallas_call(
        body,
        grid=(BATCH // blk,),
        in_specs=[
            pl.BlockSpec((blk, EMBED_DIM), lambda i: (i, 0)),
            pl.BlockSpec((blk, SEQ_PAD), lambda i: (i, 0)),
            pl.BlockSpec((1, EMBED_DIM), lambda i: (0, 0)),
            pl.BlockSpec((EMBED_DIM, CLASS_PAD), lambda i: (0, 0)),
            pl.BlockSpec((1, CLASS_PAD), lambda i: (0, 0)),
        ],
        out_specs=pl.BlockSpec((blk, CLASS_PAD), lambda i: (i, 0)),
        out_shape=jax.ShapeDtypeStruct((BATCH, CLASS_PAD), jnp.float32),
    )(sums, x_pad, t0, w_scaled, b_pad)


def kernel(x, table, W, b):
    x_pad = jnp.pad(x, ((0, 0), (0, SEQ_PAD - SEQ)))
    t0 = lax.slice(table, (0, 0), (1, EMBED_DIM))
    w_scaled = jnp.zeros((EMBED_DIM, CLASS_PAD), jnp.float32)
    w_scaled = w_scaled.at[:, :NUM_CLASSES].set(W.T * (1.0 / SEQ))
    b_pad = jnp.zeros((1, CLASS_PAD), jnp.float32).at[0, :NUM_CLASSES].set(b)

    sums = _sc_pool_sums(x_pad, table)
    logits_pad = _tc_matmul(sums, x_pad, t0, w_scaled, b_pad)
    return logits_pad[:, :NUM_CLASSES]
